# Initial kernel scaffold; baseline (speedup 1.0000x reference)
import jax
import jax.numpy as jnp
from jax.experimental import pallas as pl


def kernel(x, edge_index, W1, b1, W2, b2):
    raise NotImplementedError("write your pallas kernel here")


# speedup vs baseline: 19.9407x; 19.9407x over previous
import functools

import jax
import jax.numpy as jnp
from jax import lax
from jax.experimental import pallas as pl
from jax.experimental.pallas import tpu as pltpu
from jax.experimental.pallas import tpu_sc as plsc

NC = 2
NS = 16
K = 128


def _mesh():
    return plsc.VectorSubcoreMesh(
        core_axis_name="c", subcore_axis_name="s", num_cores=NC, num_subcores=NS
    )


def _sc_degree(dst_tiles, zeros_col, n_pad, n_chunks):
    rpt = n_pad // NS

    @functools.partial(
        pl.kernel,
        mesh=_mesh(),
        compiler_params=pltpu.CompilerParams(use_tc_tiling_on_sc=False),
        out_type=jax.ShapeDtypeStruct((NC, n_pad), jnp.float32),
        scratch_types=[
            pltpu.VMEM((n_chunks, K), jnp.int32),
            pltpu.VMEM((K,), jnp.float32),
            pltpu.VMEM_SHARED((n_pad,), jnp.float32),
        ],
    )
    def deg_kernel(dst_hbm, z_hbm, out_hbm, idx_d, half_v, acc):
        c = lax.axis_index("c")
        s = lax.axis_index("s")
        pltpu.sync_copy(dst_hbm.at[s], idx_d)
        for i in range(K // 16):
            half_v[pl.ds(16 * i, 16)] = jnp.full((16,), 0.5, jnp.float32)
        pltpu.sync_copy(z_hbm.at[pl.ds(s * rpt, rpt)], acc.at[pl.ds(s * rpt, rpt)])
        plsc.subcore_barrier()

        def body(g, carry):
            pltpu.sync_copy(half_v, acc.at[idx_d.at[g]], add=True)
            return carry

        lax.fori_loop(0, n_chunks, body, 0)
        plsc.subcore_barrier()
        pltpu.sync_copy(acc.at[pl.ds(s * rpt, rpt)], out_hbm.at[c, pl.ds(s * rpt, rpt)])

    return deg_kernel(dst_tiles, zeros_col)


def _sc_aggregate(h_split, src_tiles, dst_tiles, zeros_mat, n_pad, n_chunks):
    d2 = h_split.shape[2]
    rpt = n_pad // NS

    @functools.partial(
        pl.kernel,
        mesh=_mesh(),
        compiler_params=pltpu.CompilerParams(use_tc_tiling_on_sc=False),
        out_type=jax.ShapeDtypeStruct((NC, n_pad, d2), jnp.float32),
        scratch_types=[
            pltpu.VMEM((n_chunks, K), jnp.int32),
            pltpu.VMEM((n_chunks, K), jnp.int32),
            pltpu.VMEM((2, K, d2), jnp.float32),
            pltpu.VMEM_SHARED((n_pad, d2), jnp.float32),
            pltpu.SemaphoreType.DMA,
        ],
    )
    def agg_kernel(h_hbm, src_hbm, dst_hbm, z_hbm, out_hbm, idx_s, idx_d, rows, acc, gsem):
        c = lax.axis_index("c")
        s = lax.axis_index("s")
        pltpu.sync_copy(src_hbm.at[s], idx_s)
        pltpu.sync_copy(dst_hbm.at[s], idx_d)
        pltpu.sync_copy(z_hbm.at[pl.ds(s * rpt, rpt)], acc.at[pl.ds(s * rpt, rpt)])
        plsc.subcore_barrier()
        pltpu.async_copy(h_hbm.at[c].at[idx_s.at[0]], rows.at[0], gsem)

        def body(g, carry):
            slot = lax.rem(g, 2)
            pltpu.make_async_copy(h_hbm.at[c].at[idx_s.at[g]], rows.at[slot], gsem).wait()

            @pl.when(g + 1 < n_chunks)
            def _prefetch():
                pltpu.async_copy(
                    h_hbm.at[c].at[idx_s.at[g + 1]], rows.at[lax.rem(g + 1, 2)], gsem
                )

            pltpu.sync_copy(rows.at[slot], acc.at[idx_d.at[g]], add=True)
            return carry

        lax.fori_loop(0, n_chunks, body, 0)
        plsc.subcore_barrier()
        pltpu.sync_copy(acc.at[pl.ds(s * rpt, rpt)], out_hbm.at[c, pl.ds(s * rpt, rpt)])

    return agg_kernel(h_split, src_tiles, dst_tiles, zeros_mat)


def _tc_dis(d0, d1):
    def body(a_ref, b_ref, o_ref):
        o_ref[...] = lax.rsqrt(1.0 + a_ref[...] + b_ref[...])

    return pl.pallas_call(
        body, out_shape=jax.ShapeDtypeStruct(d0.shape, jnp.float32)
    )(d0, d1)


def _tc_mm_scale(x, ws, discol, blk):
    n, din = x.shape
    d2 = ws.shape[2]

    def body(x_ref, w_ref, s_ref, o_ref):
        o_ref[0] = s_ref[...] * jnp.dot(
            x_ref[...], w_ref[0], preferred_element_type=jnp.float32
        )

    return pl.pallas_call(
        body,
        grid=(NC, n // blk),
        in_specs=[
            pl.BlockSpec((blk, din), lambda t, i: (i, 0)),
            pl.BlockSpec((1, din, d2), lambda t, i: (t, 0, 0)),
            pl.BlockSpec((blk, 1), lambda t, i: (i, 0)),
        ],
        out_specs=pl.BlockSpec((1, blk, d2), lambda t, i: (t, i, 0)),
        out_shape=jax.ShapeDtypeStruct((NC, n, d2), jnp.float32),
    )(x, ws, discol)


def _tc_mid(acc, ht, discol, b, w2s, blk):
    nc, n, d2 = ht.shape
    d = nc * d2
    do2 = w2s.shape[2]

    def body(a_ref, h_ref, s_ref, b_ref, w_ref, o_ref):
        agg = jnp.concatenate([a_ref[0] + h_ref[0], a_ref[1] + h_ref[1]], axis=-1)
        h1 = jnp.maximum(s_ref[...] * agg + b_ref[...], 0.0)
        o_ref[0] = s_ref[...] * jnp.dot(
            h1, w_ref[0], preferred_element_type=jnp.float32
        )

    return pl.pallas_call(
        body,
        grid=(NC, n // blk),
        in_specs=[
            pl.BlockSpec((NC, blk, d2), lambda t, i: (0, i, 0)),
            pl.BlockSpec((NC, blk, d2), lambda t, i: (0, i, 0)),
            pl.BlockSpec((blk, 1), lambda t, i: (i, 0)),
            pl.BlockSpec((1, d), lambda t, i: (0, 0)),
            pl.BlockSpec((1, d, do2), lambda t, i: (t, 0, 0)),
        ],
        out_specs=pl.BlockSpec((1, blk, do2), lambda t, i: (t, i, 0)),
        out_shape=jax.ShapeDtypeStruct((NC, n, do2), jnp.float32),
    )(acc, ht, discol, b, w2s)


def _tc_post(acc, ht, discol, b, blk):
    nc, n, d2 = ht.shape
    d = nc * d2

    def body(a_ref, h_ref, s_ref, b_ref, o_ref):
        agg = jnp.concatenate([a_ref[0] + h_ref[0], a_ref[1] + h_ref[1]], axis=-1)
        o_ref[...] = s_ref[...] * agg + b_ref[...]

    return pl.pallas_call(
        body,
        grid=(n // blk,),
        in_specs=[
            pl.BlockSpec((NC, blk, d2), lambda i: (0, i, 0)),
            pl.BlockSpec((NC, blk, d2), lambda i: (0, i, 0)),
            pl.BlockSpec((blk, 1), lambda i: (i, 0)),
            pl.BlockSpec((1, d), lambda i: (0, 0)),
        ],
        out_specs=pl.BlockSpec((blk, d), lambda i: (i, 0)),
        out_shape=jax.ShapeDtypeStruct((n, d), jnp.float32),
    )(acc, ht, discol, b)


def kernel(x, edge_index, W1, b1, W2, b2):
    n, din = x.shape
    e = edge_index.shape[1]

    epw = K * (-(-e // (K * NS)))
    n_chunks = epw // K
    e_pad = epw * NS
    n_pad = 128 * NS * (-(-(n + 1) // (128 * NS)))

    ei = edge_index.astype(jnp.int32)
    pad = e_pad - e
    src_t = jnp.concatenate([ei[0], jnp.zeros((pad,), jnp.int32)]).reshape(
        NS, n_chunks, K
    )
    dst_t = jnp.concatenate([ei[1], jnp.full((pad,), n, jnp.int32)]).reshape(
        NS, n_chunks, K
    )
    z_col = jnp.zeros((n_pad,), jnp.float32)
    z_hid = jnp.zeros((n_pad, W1.shape[1] // NC), jnp.float32)
    z_out = jnp.zeros((n_pad, W2.shape[1] // NC), jnp.float32)

    blk = 2000 if n % 2000 == 0 else n

    h2 = W1.shape[1] // NC
    o2 = W2.shape[1] // NC
    w1s = jnp.stack([W1[:, :h2], W1[:, h2:]])
    w2s = jnp.stack([W2[:, :o2], W2[:, o2:]])

    deg = _sc_degree(dst_t, z_col, n_pad, n_chunks)
    dis2 = _tc_dis(deg[0].reshape(-1, 128), deg[1].reshape(-1, 128))
    discol = dis2.reshape(-1)[:n][:, None]

    ht1 = _tc_mm_scale(x, w1s, discol, blk)
    acc1 = _sc_aggregate(ht1, src_t, dst_t, z_hid, n_pad, n_chunks)
    ht2 = _tc_mid(acc1[:, :n], ht1, discol, b1[None, :], w2s, blk)
    acc2 = _sc_aggregate(ht2, src_t, dst_t, z_out, n_pad, n_chunks)
    return _tc_post(acc2[:, :n], ht2, discol, b2[None, :], blk)


# speedup vs baseline: 27.3834x; 1.3732x over previous
import functools

import jax
import jax.numpy as jnp
from jax import lax
from jax.experimental import pallas as pl
from jax.experimental.pallas import tpu as pltpu
from jax.experimental.pallas import tpu_sc as plsc

NC = 2
NS = 16
K = 128


def _mesh():
    return plsc.VectorSubcoreMesh(
        core_axis_name="c", subcore_axis_name="s", num_cores=NC, num_subcores=NS
    )


def _sc_degree(dst_tiles, zeros_col, n_pad, n_chunks):
    rpt = n_pad // NS

    @functools.partial(
        pl.kernel,
        mesh=_mesh(),
        compiler_params=pltpu.CompilerParams(use_tc_tiling_on_sc=False),
        out_type=jax.ShapeDtypeStruct((NC, n_pad), jnp.float32),
        scratch_types=[
            pltpu.VMEM((n_chunks, K), jnp.int32),
            pltpu.VMEM((K,), jnp.float32),
            pltpu.VMEM_SHARED((n_pad,), jnp.float32),
        ],
    )
    def deg_kernel(dst_hbm, z_hbm, out_hbm, idx_d, half_v, acc):
        c = lax.axis_index("c")
        s = lax.axis_index("s")
        pltpu.sync_copy(dst_hbm.at[s], idx_d)
        for i in range(K // 16):
            half_v[pl.ds(16 * i, 16)] = jnp.full((16,), 0.5, jnp.float32)
        pltpu.sync_copy(z_hbm.at[pl.ds(s * rpt, rpt)], acc.at[pl.ds(s * rpt, rpt)])
        plsc.subcore_barrier()

        def body(g, carry):
            pltpu.sync_copy(half_v, acc.at[idx_d.at[g]], add=True)
            return carry

        lax.fori_loop(0, n_chunks, body, 0)
        plsc.subcore_barrier()
        pltpu.sync_copy(acc.at[pl.ds(s * rpt, rpt)], out_hbm.at[c, pl.ds(s * rpt, rpt)])

    return deg_kernel(dst_tiles, zeros_col)


def _sc_aggregate(h_split, src_tiles, dst_tiles, zeros_mat, n_pad, n_chunks):
    d2 = h_split.shape[2]
    rpt = n_pad // NS

    @functools.partial(
        pl.kernel,
        mesh=_mesh(),
        compiler_params=pltpu.CompilerParams(use_tc_tiling_on_sc=False),
        out_type=jax.ShapeDtypeStruct((NC, n_pad, d2), jnp.float32),
        scratch_types=[
            pltpu.VMEM((n_chunks, K), jnp.int32),
            pltpu.VMEM((n_chunks, K), jnp.int32),
            pltpu.VMEM((4, K, d2), jnp.float32),
            pltpu.VMEM_SHARED((n_pad, d2), jnp.float32),
            pltpu.SemaphoreType.DMA,
            pltpu.SemaphoreType.DMA,
        ],
    )
    def agg_kernel(h_hbm, src_hbm, dst_hbm, z_hbm, out_hbm, idx_s, idx_d, rows, acc, gsem, ssem):
        c = lax.axis_index("c")
        s = lax.axis_index("s")
        pltpu.sync_copy(src_hbm.at[s], idx_s)
        pltpu.sync_copy(dst_hbm.at[s], idx_d)
        pltpu.sync_copy(z_hbm.at[pl.ds(s * rpt, rpt)], acc.at[pl.ds(s * rpt, rpt)])
        plsc.subcore_barrier()
        for j in range(3):
            pltpu.async_copy(h_hbm.at[c].at[idx_s.at[j]], rows.at[j], gsem)

        def body(g, carry):
            slot = lax.rem(g, 4)
            pltpu.make_async_copy(h_hbm.at[c].at[idx_s.at[g]], rows.at[slot], gsem).wait()
            pltpu.async_copy(rows.at[slot], acc.at[idx_d.at[g]], ssem, add=True)

            @pl.when(g >= 1)
            def _drain_prev():
                pltpu.make_async_copy(
                    rows.at[lax.rem(g + 3, 4)], acc.at[idx_d.at[g - 1]], ssem
                ).wait()

            @pl.when(g + 3 < n_chunks)
            def _prefetch():
                pltpu.async_copy(
                    h_hbm.at[c].at[idx_s.at[g + 3]], rows.at[lax.rem(g + 3, 4)], gsem
                )

            return carry

        lax.fori_loop(0, n_chunks, body, 0)
        last = n_chunks - 1
        pltpu.make_async_copy(
            rows.at[lax.rem(last, 4)], acc.at[idx_d.at[last]], ssem
        ).wait()
        plsc.subcore_barrier()
        pltpu.sync_copy(acc.at[pl.ds(s * rpt, rpt)], out_hbm.at[c, pl.ds(s * rpt, rpt)])

    return agg_kernel(h_split, src_tiles, dst_tiles, zeros_mat)


def _tc_mm_scale(x, ws, d0, d1, blk):
    n, din = x.shape
    d2 = ws.shape[2]

    def body(x_ref, w_ref, a_ref, b_ref, o_ref, s_ref):
        dis = lax.rsqrt(1.0 + a_ref[...] + b_ref[...])
        s_ref[...] = dis
        o_ref[0] = dis * jnp.dot(
            x_ref[...], w_ref[0], preferred_element_type=jnp.float32
        )

    return pl.pallas_call(
        body,
        grid=(NC, n // blk),
        in_specs=[
            pl.BlockSpec((blk, din), lambda t, i: (i, 0)),
            pl.BlockSpec((1, din, d2), lambda t, i: (t, 0, 0)),
            pl.BlockSpec((blk, 1), lambda t, i: (i, 0)),
            pl.BlockSpec((blk, 1), lambda t, i: (i, 0)),
        ],
        out_specs=[
            pl.BlockSpec((1, blk, d2), lambda t, i: (t, i, 0)),
            pl.BlockSpec((blk, 1), lambda t, i: (i, 0)),
        ],
        out_shape=[
            jax.ShapeDtypeStruct((NC, n, d2), jnp.float32),
            jax.ShapeDtypeStruct((n, 1), jnp.float32),
        ],
    )(x, ws, d0, d1)


def _tc_mid(acc, ht, discol, b, w2s, blk):
    nc, n, d2 = ht.shape
    d = nc * d2
    do2 = w2s.shape[2]

    def body(a_ref, h_ref, s_ref, b_ref, w_ref, o_ref):
        agg = jnp.concatenate([a_ref[0] + h_ref[0], a_ref[1] + h_ref[1]], axis=-1)
        h1 = jnp.maximum(s_ref[...] * agg + b_ref[...], 0.0)
        o_ref[0] = s_ref[...] * jnp.dot(
            h1, w_ref[0], preferred_element_type=jnp.float32
        )

    return pl.pallas_call(
        body,
        grid=(NC, n // blk),
        in_specs=[
            pl.BlockSpec((NC, blk, d2), lambda t, i: (0, i, 0)),
            pl.BlockSpec((NC, blk, d2), lambda t, i: (0, i, 0)),
            pl.BlockSpec((blk, 1), lambda t, i: (i, 0)),
            pl.BlockSpec((1, d), lambda t, i: (0, 0)),
            pl.BlockSpec((1, d, do2), lambda t, i: (t, 0, 0)),
        ],
        out_specs=pl.BlockSpec((1, blk, do2), lambda t, i: (t, i, 0)),
        out_shape=jax.ShapeDtypeStruct((NC, n, do2), jnp.float32),
    )(acc, ht, discol, b, w2s)


def _tc_post(acc, ht, discol, b, blk):
    nc, n, d2 = ht.shape
    d = nc * d2

    def body(a_ref, h_ref, s_ref, b_ref, o_ref):
        agg = jnp.concatenate([a_ref[0] + h_ref[0], a_ref[1] + h_ref[1]], axis=-1)
        o_ref[...] = s_ref[...] * agg + b_ref[...]

    return pl.pallas_call(
        body,
        grid=(n // blk,),
        in_specs=[
            pl.BlockSpec((NC, blk, d2), lambda i: (0, i, 0)),
            pl.BlockSpec((NC, blk, d2), lambda i: (0, i, 0)),
            pl.BlockSpec((blk, 1), lambda i: (i, 0)),
            pl.BlockSpec((1, d), lambda i: (0, 0)),
        ],
        out_specs=pl.BlockSpec((blk, d), lambda i: (i, 0)),
        out_shape=jax.ShapeDtypeStruct((n, d), jnp.float32),
    )(acc, ht, discol, b)


def kernel(x, edge_index, W1, b1, W2, b2):
    n, din = x.shape
    e = edge_index.shape[1]

    epw = K * (-(-e // (K * NS)))
    n_chunks = epw // K
    e_pad = epw * NS
    n_pad = 128 * NS * (-(-(n + 1) // (128 * NS)))

    ei = edge_index.astype(jnp.int32)
    pad = e_pad - e
    src_t = jnp.concatenate([ei[0], jnp.zeros((pad,), jnp.int32)]).reshape(
        NS, n_chunks, K
    )
    dst_t = jnp.concatenate([ei[1], jnp.full((pad,), n, jnp.int32)]).reshape(
        NS, n_chunks, K
    )
    z_col = jnp.zeros((n_pad,), jnp.float32)
    z_hid = jnp.zeros((n_pad, W1.shape[1] // NC), jnp.float32)
    z_out = jnp.zeros((n_pad, W2.shape[1] // NC), jnp.float32)

    blk = 2000 if n % 2000 == 0 else n

    h2 = W1.shape[1] // NC
    o2 = W2.shape[1] // NC
    w1s = jnp.stack([W1[:, :h2], W1[:, h2:]])
    w2s = jnp.stack([W2[:, :o2], W2[:, o2:]])

    deg = _sc_degree(dst_t, z_col, n_pad, n_chunks)
    ht1, discol = _tc_mm_scale(
        x, w1s, deg[0, :n, None], deg[1, :n, None], blk
    )
    acc1 = _sc_aggregate(ht1, src_t, dst_t, z_hid, n_pad, n_chunks)
    ht2 = _tc_mid(acc1[:, :n], ht1, discol, b1[None, :], w2s, blk)
    acc2 = _sc_aggregate(ht2, src_t, dst_t, z_out, n_pad, n_chunks)
    return _tc_post(acc2[:, :n], ht2, discol, b2[None, :], blk)


# speedup vs baseline: 28.2679x; 1.0323x over previous
import functools

import jax
import jax.numpy as jnp
from jax import lax
from jax.experimental import pallas as pl
from jax.experimental.pallas import tpu as pltpu
from jax.experimental.pallas import tpu_sc as plsc

NC = 2
NS = 16
K = 128


def _mesh():
    return plsc.VectorSubcoreMesh(
        core_axis_name="c", subcore_axis_name="s", num_cores=NC, num_subcores=NS
    )


def _sc_degree(dst_tiles, zeros_col, n_pad, n_chunks):
    rpt = n_pad // NS

    @functools.partial(
        pl.kernel,
        mesh=_mesh(),
        compiler_params=pltpu.CompilerParams(use_tc_tiling_on_sc=False),
        out_type=jax.ShapeDtypeStruct((NC, n_pad), jnp.float32),
        scratch_types=[
            pltpu.VMEM((n_chunks, K), jnp.int32),
            pltpu.VMEM((K,), jnp.float32),
            pltpu.VMEM_SHARED((n_pad,), jnp.float32),
        ],
    )
    def deg_kernel(dst_hbm, z_hbm, out_hbm, idx_d, half_v, acc):
        c = lax.axis_index("c")
        s = lax.axis_index("s")
        pltpu.sync_copy(dst_hbm.at[s], idx_d)
        for i in range(K // 16):
            half_v[pl.ds(16 * i, 16)] = jnp.full((16,), 0.5, jnp.float32)
        pltpu.sync_copy(z_hbm.at[pl.ds(s * rpt, rpt)], acc.at[pl.ds(s * rpt, rpt)])
        plsc.subcore_barrier()

        def body(g, carry):
            pltpu.sync_copy(half_v, acc.at[idx_d.at[g]], add=True)
            return carry

        lax.fori_loop(0, n_chunks, body, 0)
        plsc.subcore_barrier()
        pltpu.sync_copy(acc.at[pl.ds(s * rpt, rpt)], out_hbm.at[c, pl.ds(s * rpt, rpt)])

    return deg_kernel(dst_tiles, zeros_col)


def _sc_aggregate(h_split, src_tiles, dst_tiles, zeros_mat, n_pad, n_chunks):
    d2 = h_split.shape[2]
    rpt = n_pad // NS
    nbuf = 8 if d2 <= 32 else 6

    @functools.partial(
        pl.kernel,
        mesh=_mesh(),
        compiler_params=pltpu.CompilerParams(use_tc_tiling_on_sc=False),
        out_type=jax.ShapeDtypeStruct((NC, n_pad, d2), jnp.float32),
        scratch_types=[
            pltpu.VMEM((n_chunks, K), jnp.int32),
            pltpu.VMEM((n_chunks, K), jnp.int32),
            pltpu.VMEM((nbuf, K, d2), jnp.float32),
            pltpu.VMEM_SHARED((n_pad, d2), jnp.float32),
            pltpu.SemaphoreType.DMA,
            pltpu.SemaphoreType.DMA,
            pltpu.SemaphoreType.DMA,
        ],
    )
    def agg_kernel(h_hbm, src_hbm, dst_hbm, z_hbm, out_hbm, idx_s, idx_d, rows, acc, gsem, ssem, isem):
        c = lax.axis_index("c")
        s = lax.axis_index("s")
        pltpu.async_copy(src_hbm.at[s], idx_s, isem)
        pltpu.async_copy(dst_hbm.at[s], idx_d, isem)
        pltpu.async_copy(
            z_hbm.at[pl.ds(s * rpt, rpt)], acc.at[pl.ds(s * rpt, rpt)], isem
        )
        pltpu.make_async_copy(src_hbm.at[s], idx_s, isem).wait()
        pltpu.make_async_copy(dst_hbm.at[s], idx_d, isem).wait()
        pltpu.make_async_copy(
            z_hbm.at[pl.ds(s * rpt, rpt)], acc.at[pl.ds(s * rpt, rpt)], isem
        ).wait()
        plsc.subcore_barrier()
        for j in range(nbuf - 1):
            pltpu.async_copy(h_hbm.at[c].at[idx_s.at[j]], rows.at[j], gsem)

        def body(g, carry):
            slot = lax.rem(g, nbuf)
            pltpu.make_async_copy(h_hbm.at[c].at[idx_s.at[g]], rows.at[slot], gsem).wait()
            pltpu.async_copy(rows.at[slot], acc.at[idx_d.at[g]], ssem, add=True)

            @pl.when(g >= 1)
            def _drain_prev():
                pltpu.make_async_copy(
                    rows.at[lax.rem(g + nbuf - 1, nbuf)], acc.at[idx_d.at[g - 1]], ssem
                ).wait()

            @pl.when(g + nbuf - 1 < n_chunks)
            def _prefetch():
                pltpu.async_copy(
                    h_hbm.at[c].at[idx_s.at[g + nbuf - 1]],
                    rows.at[lax.rem(g + nbuf - 1, nbuf)],
                    gsem,
                )

            return carry

        lax.fori_loop(0, n_chunks, body, 0)
        last = n_chunks - 1
        pltpu.make_async_copy(
            rows.at[lax.rem(last, nbuf)], acc.at[idx_d.at[last]], ssem
        ).wait()
        plsc.subcore_barrier()
        pltpu.sync_copy(acc.at[pl.ds(s * rpt, rpt)], out_hbm.at[c, pl.ds(s * rpt, rpt)])

    return agg_kernel(h_split, src_tiles, dst_tiles, zeros_mat)


def _tc_mm_scale(x, ws, d0, d1, blk):
    n, din = x.shape
    d2 = ws.shape[2]

    def body(x_ref, w_ref, a_ref, b_ref, o_ref, s_ref):
        dis = lax.rsqrt(1.0 + a_ref[...] + b_ref[...])
        s_ref[...] = dis
        o_ref[0] = dis * jnp.dot(
            x_ref[...], w_ref[0], preferred_element_type=jnp.float32
        )

    return pl.pallas_call(
        body,
        grid=(NC, n // blk),
        in_specs=[
            pl.BlockSpec((blk, din), lambda t, i: (i, 0)),
            pl.BlockSpec((1, din, d2), lambda t, i: (t, 0, 0)),
            pl.BlockSpec((blk, 1), lambda t, i: (i, 0)),
            pl.BlockSpec((blk, 1), lambda t, i: (i, 0)),
        ],
        out_specs=[
            pl.BlockSpec((1, blk, d2), lambda t, i: (t, i, 0)),
            pl.BlockSpec((blk, 1), lambda t, i: (i, 0)),
        ],
        out_shape=[
            jax.ShapeDtypeStruct((NC, n, d2), jnp.float32),
            jax.ShapeDtypeStruct((n, 1), jnp.float32),
        ],
    )(x, ws, d0, d1)


def _tc_mid(acc, ht, discol, b, w2s, blk):
    nc, n, d2 = ht.shape
    d = nc * d2
    do2 = w2s.shape[2]

    def body(a_ref, h_ref, s_ref, b_ref, w_ref, o_ref):
        agg = jnp.concatenate([a_ref[0] + h_ref[0], a_ref[1] + h_ref[1]], axis=-1)
        h1 = jnp.maximum(s_ref[...] * agg + b_ref[...], 0.0)
        o_ref[0] = s_ref[...] * jnp.dot(
            h1, w_ref[0], preferred_element_type=jnp.float32
        )

    return pl.pallas_call(
        body,
        grid=(NC, n // blk),
        in_specs=[
            pl.BlockSpec((NC, blk, d2), lambda t, i: (0, i, 0)),
            pl.BlockSpec((NC, blk, d2), lambda t, i: (0, i, 0)),
            pl.BlockSpec((blk, 1), lambda t, i: (i, 0)),
            pl.BlockSpec((1, d), lambda t, i: (0, 0)),
            pl.BlockSpec((1, d, do2), lambda t, i: (t, 0, 0)),
        ],
        out_specs=pl.BlockSpec((1, blk, do2), lambda t, i: (t, i, 0)),
        out_shape=jax.ShapeDtypeStruct((NC, n, do2), jnp.float32),
    )(acc, ht, discol, b, w2s)


def _tc_post(acc, ht, discol, b, blk):
    nc, n, d2 = ht.shape
    d = nc * d2

    def body(a_ref, h_ref, s_ref, b_ref, o_ref):
        agg = jnp.concatenate([a_ref[0] + h_ref[0], a_ref[1] + h_ref[1]], axis=-1)
        o_ref[...] = s_ref[...] * agg + b_ref[...]

    return pl.pallas_call(
        body,
        grid=(n // blk,),
        in_specs=[
            pl.BlockSpec((NC, blk, d2), lambda i: (0, i, 0)),
            pl.BlockSpec((NC, blk, d2), lambda i: (0, i, 0)),
            pl.BlockSpec((blk, 1), lambda i: (i, 0)),
            pl.BlockSpec((1, d), lambda i: (0, 0)),
        ],
        out_specs=pl.BlockSpec((blk, d), lambda i: (i, 0)),
        out_shape=jax.ShapeDtypeStruct((n, d), jnp.float32),
    )(acc, ht, discol, b)


def kernel(x, edge_index, W1, b1, W2, b2):
    n, din = x.shape
    e = edge_index.shape[1]

    epw = K * (-(-e // (K * NS)))
    n_chunks = epw // K
    e_pad = epw * NS
    n_pad = 128 * NS * (-(-(n + 1) // (128 * NS)))

    ei = edge_index.astype(jnp.int32)
    pad = e_pad - e
    src_t = jnp.concatenate([ei[0], jnp.zeros((pad,), jnp.int32)]).reshape(
        NS, n_chunks, K
    )
    dst_t = jnp.concatenate([ei[1], jnp.full((pad,), n, jnp.int32)]).reshape(
        NS, n_chunks, K
    )
    z_col = jnp.zeros((n_pad,), jnp.float32)
    z_hid = jnp.zeros((n_pad, W1.shape[1] // NC), jnp.float32)
    z_out = jnp.zeros((n_pad, W2.shape[1] // NC), jnp.float32)

    blk = 2000 if n % 2000 == 0 else n

    h2 = W1.shape[1] // NC
    o2 = W2.shape[1] // NC
    w1s = jnp.stack([W1[:, :h2], W1[:, h2:]])
    w2s = jnp.stack([W2[:, :o2], W2[:, o2:]])

    deg = _sc_degree(dst_t, z_col, n_pad, n_chunks)
    ht1, discol = _tc_mm_scale(
        x, w1s, deg[0, :n, None], deg[1, :n, None], blk
    )
    acc1 = _sc_aggregate(ht1, src_t, dst_t, z_hid, n_pad, n_chunks)
    ht2 = _tc_mid(acc1[:, :n], ht1, discol, b1[None, :], w2s, blk)
    acc2 = _sc_aggregate(ht2, src_t, dst_t, z_out, n_pad, n_chunks)
    return _tc_post(acc2[:, :n], ht2, discol, b2[None, :], blk)


# speedup vs baseline: 28.2866x; 1.0007x over previous
import functools

import jax
import jax.numpy as jnp
from jax import lax
from jax.experimental import pallas as pl
from jax.experimental.pallas import tpu as pltpu
from jax.experimental.pallas import tpu_sc as plsc

NC = 2
NS = 16
K = 128


def _mesh():
    return plsc.VectorSubcoreMesh(
        core_axis_name="c", subcore_axis_name="s", num_cores=NC, num_subcores=NS
    )


def _sc_degree(dst_tiles, zeros_col, n_pad, n_chunks):
    rpt = n_pad // NS

    @functools.partial(
        pl.kernel,
        mesh=_mesh(),
        compiler_params=pltpu.CompilerParams(use_tc_tiling_on_sc=False),
        out_type=jax.ShapeDtypeStruct((NC, n_pad), jnp.float32),
        scratch_types=[
            pltpu.VMEM((n_chunks, K), jnp.int32),
            pltpu.VMEM((K,), jnp.float32),
            pltpu.VMEM_SHARED((n_pad,), jnp.float32),
        ],
    )
    def deg_kernel(dst_hbm, z_hbm, out_hbm, idx_d, half_v, acc):
        c = lax.axis_index("c")
        s = lax.axis_index("s")
        pltpu.sync_copy(dst_hbm.at[s], idx_d)
        for i in range(K // 16):
            half_v[pl.ds(16 * i, 16)] = jnp.full((16,), 0.5, jnp.float32)
        pltpu.sync_copy(z_hbm.at[pl.ds(s * rpt, rpt)], acc.at[pl.ds(s * rpt, rpt)])
        plsc.subcore_barrier()

        def body(g, carry):
            pltpu.sync_copy(half_v, acc.at[idx_d.at[g]], add=True)
            return carry

        lax.fori_loop(0, n_chunks, body, 0)
        plsc.subcore_barrier()
        pltpu.sync_copy(acc.at[pl.ds(s * rpt, rpt)], out_hbm.at[c, pl.ds(s * rpt, rpt)])

    return deg_kernel(dst_tiles, zeros_col)


def _sc_aggregate(h_split, src_tiles, dst_tiles, zeros_mat, n_pad, n_chunks):
    d2 = h_split.shape[2]
    rpt = n_pad // NS
    nbuf = 8 if d2 <= 32 else 6
    ga = nbuf - 3
    ss = 2

    @functools.partial(
        pl.kernel,
        mesh=_mesh(),
        compiler_params=pltpu.CompilerParams(use_tc_tiling_on_sc=False),
        out_type=jax.ShapeDtypeStruct((NC, n_pad, d2), jnp.float32),
        scratch_types=[
            pltpu.VMEM((n_chunks, K), jnp.int32),
            pltpu.VMEM((n_chunks, K), jnp.int32),
            pltpu.VMEM((nbuf, K, d2), jnp.float32),
            pltpu.VMEM_SHARED((n_pad, d2), jnp.float32),
            pltpu.SemaphoreType.DMA,
            pltpu.SemaphoreType.DMA,
            pltpu.SemaphoreType.DMA,
        ],
    )
    def agg_kernel(h_hbm, src_hbm, dst_hbm, z_hbm, out_hbm, idx_s, idx_d, rows, acc, gsem, ssem, isem):
        c = lax.axis_index("c")
        s = lax.axis_index("s")
        pltpu.async_copy(src_hbm.at[s], idx_s, isem)
        pltpu.async_copy(dst_hbm.at[s], idx_d, isem)
        pltpu.async_copy(
            z_hbm.at[pl.ds(s * rpt, rpt)], acc.at[pl.ds(s * rpt, rpt)], isem
        )
        pltpu.make_async_copy(src_hbm.at[s], idx_s, isem).wait()
        pltpu.make_async_copy(dst_hbm.at[s], idx_d, isem).wait()
        pltpu.make_async_copy(
            z_hbm.at[pl.ds(s * rpt, rpt)], acc.at[pl.ds(s * rpt, rpt)], isem
        ).wait()
        plsc.subcore_barrier()
        for j in range(ga):
            pltpu.async_copy(h_hbm.at[c].at[idx_s.at[j]], rows.at[j], gsem)

        def body(g, carry):
            slot = lax.rem(g, nbuf)
            pltpu.make_async_copy(h_hbm.at[c].at[idx_s.at[g]], rows.at[slot], gsem).wait()
            pltpu.async_copy(rows.at[slot], acc.at[idx_d.at[g]], ssem, add=True)

            @pl.when(g >= ss)
            def _drain_prev():
                pltpu.make_async_copy(
                    rows.at[lax.rem(g - ss, nbuf)], acc.at[idx_d.at[g - ss]], ssem
                ).wait()

            @pl.when(g + ga < n_chunks)
            def _prefetch():
                pltpu.async_copy(
                    h_hbm.at[c].at[idx_s.at[g + ga]],
                    rows.at[lax.rem(g + ga, nbuf)],
                    gsem,
                )

            return carry

        lax.fori_loop(0, n_chunks, body, 0)
        for j in range(n_chunks - ss, n_chunks):
            pltpu.make_async_copy(
                rows.at[j % nbuf], acc.at[idx_d.at[j]], ssem
            ).wait()
        plsc.subcore_barrier()
        pltpu.sync_copy(acc.at[pl.ds(s * rpt, rpt)], out_hbm.at[c, pl.ds(s * rpt, rpt)])

    return agg_kernel(h_split, src_tiles, dst_tiles, zeros_mat)


def _tc_mm_scale(x, ws, d0, d1, blk):
    n, din = x.shape
    d2 = ws.shape[2]

    def body(x_ref, w_ref, a_ref, b_ref, o_ref, s_ref):
        dis = lax.rsqrt(1.0 + a_ref[...] + b_ref[...])
        s_ref[...] = dis
        o_ref[0] = dis * jnp.dot(
            x_ref[...], w_ref[0], preferred_element_type=jnp.float32
        )

    return pl.pallas_call(
        body,
        grid=(NC, n // blk),
        in_specs=[
            pl.BlockSpec((blk, din), lambda t, i: (i, 0)),
            pl.BlockSpec((1, din, d2), lambda t, i: (t, 0, 0)),
            pl.BlockSpec((blk, 1), lambda t, i: (i, 0)),
            pl.BlockSpec((blk, 1), lambda t, i: (i, 0)),
        ],
        out_specs=[
            pl.BlockSpec((1, blk, d2), lambda t, i: (t, i, 0)),
            pl.BlockSpec((blk, 1), lambda t, i: (i, 0)),
        ],
        out_shape=[
            jax.ShapeDtypeStruct((NC, n, d2), jnp.float32),
            jax.ShapeDtypeStruct((n, 1), jnp.float32),
        ],
    )(x, ws, d0, d1)


def _tc_mid(acc, ht, discol, b, w2s, blk):
    nc, n, d2 = ht.shape
    d = nc * d2
    do2 = w2s.shape[2]

    def body(a_ref, h_ref, s_ref, b_ref, w_ref, o_ref):
        agg = jnp.concatenate([a_ref[0] + h_ref[0], a_ref[1] + h_ref[1]], axis=-1)
        h1 = jnp.maximum(s_ref[...] * agg + b_ref[...], 0.0)
        o_ref[0] = s_ref[...] * jnp.dot(
            h1, w_ref[0], preferred_element_type=jnp.float32
        )

    return pl.pallas_call(
        body,
        grid=(NC, n // blk),
        in_specs=[
            pl.BlockSpec((NC, blk, d2), lambda t, i: (0, i, 0)),
            pl.BlockSpec((NC, blk, d2), lambda t, i: (0, i, 0)),
            pl.BlockSpec((blk, 1), lambda t, i: (i, 0)),
            pl.BlockSpec((1, d), lambda t, i: (0, 0)),
            pl.BlockSpec((1, d, do2), lambda t, i: (t, 0, 0)),
        ],
        out_specs=pl.BlockSpec((1, blk, do2), lambda t, i: (t, i, 0)),
        out_shape=jax.ShapeDtypeStruct((NC, n, do2), jnp.float32),
    )(acc, ht, discol, b, w2s)


def _tc_post(acc, ht, discol, b, blk):
    nc, n, d2 = ht.shape
    d = nc * d2

    def body(a_ref, h_ref, s_ref, b_ref, o_ref):
        agg = jnp.concatenate([a_ref[0] + h_ref[0], a_ref[1] + h_ref[1]], axis=-1)
        o_ref[...] = s_ref[...] * agg + b_ref[...]

    return pl.pallas_call(
        body,
        grid=(n // blk,),
        in_specs=[
            pl.BlockSpec((NC, blk, d2), lambda i: (0, i, 0)),
            pl.BlockSpec((NC, blk, d2), lambda i: (0, i, 0)),
            pl.BlockSpec((blk, 1), lambda i: (i, 0)),
            pl.BlockSpec((1, d), lambda i: (0, 0)),
        ],
        out_specs=pl.BlockSpec((blk, d), lambda i: (i, 0)),
        out_shape=jax.ShapeDtypeStruct((n, d), jnp.float32),
    )(acc, ht, discol, b)


def kernel(x, edge_index, W1, b1, W2, b2):
    n, din = x.shape
    e = edge_index.shape[1]

    epw = K * (-(-e // (K * NS)))
    n_chunks = epw // K
    e_pad = epw * NS
    n_pad = 128 * NS * (-(-(n + 1) // (128 * NS)))

    ei = edge_index.astype(jnp.int32)
    pad = e_pad - e
    src_t = jnp.concatenate([ei[0], jnp.zeros((pad,), jnp.int32)]).reshape(
        NS, n_chunks, K
    )
    dst_t = jnp.concatenate([ei[1], jnp.full((pad,), n, jnp.int32)]).reshape(
        NS, n_chunks, K
    )
    z_col = jnp.zeros((n_pad,), jnp.float32)
    z_hid = jnp.zeros((n_pad, W1.shape[1] // NC), jnp.float32)
    z_out = jnp.zeros((n_pad, W2.shape[1] // NC), jnp.float32)

    blk = 2000 if n % 2000 == 0 else n

    h2 = W1.shape[1] // NC
    o2 = W2.shape[1] // NC
    w1s = jnp.stack([W1[:, :h2], W1[:, h2:]])
    w2s = jnp.stack([W2[:, :o2], W2[:, o2:]])

    deg = _sc_degree(dst_t, z_col, n_pad, n_chunks)
    ht1, discol = _tc_mm_scale(
        x, w1s, deg[0, :n, None], deg[1, :n, None], blk
    )
    acc1 = _sc_aggregate(ht1, src_t, dst_t, z_hid, n_pad, n_chunks)
    ht2 = _tc_mid(acc1[:, :n], ht1, discol, b1[None, :], w2s, blk)
    acc2 = _sc_aggregate(ht2, src_t, dst_t, z_out, n_pad, n_chunks)
    return _tc_post(acc2[:, :n], ht2, discol, b2[None, :], blk)


# speedup vs baseline: 28.3997x; 1.0040x over previous
import functools

import jax
import jax.numpy as jnp
from jax import lax
from jax.experimental import pallas as pl
from jax.experimental.pallas import tpu as pltpu
from jax.experimental.pallas import tpu_sc as plsc

NC = 2
NS = 16
K = 128


def _mesh():
    return plsc.VectorSubcoreMesh(
        core_axis_name="c", subcore_axis_name="s", num_cores=NC, num_subcores=NS
    )


def _sc_degree(dst_tiles, zeros_col, n_pad, n_chunks):
    rpt = n_pad // NS

    @functools.partial(
        pl.kernel,
        mesh=_mesh(),
        compiler_params=pltpu.CompilerParams(use_tc_tiling_on_sc=False),
        out_type=jax.ShapeDtypeStruct((NC, n_pad), jnp.float32),
        scratch_types=[
            pltpu.VMEM((n_chunks, K), jnp.int32),
            pltpu.VMEM((K,), jnp.float32),
            pltpu.VMEM_SHARED((n_pad,), jnp.float32),
        ],
    )
    def deg_kernel(dst_hbm, z_hbm, out_hbm, idx_d, half_v, acc):
        c = lax.axis_index("c")
        s = lax.axis_index("s")
        pltpu.sync_copy(dst_hbm.at[s], idx_d)
        for i in range(K // 16):
            half_v[pl.ds(16 * i, 16)] = jnp.full((16,), 0.5, jnp.float32)
        pltpu.sync_copy(z_hbm.at[pl.ds(s * rpt, rpt)], acc.at[pl.ds(s * rpt, rpt)])
        plsc.subcore_barrier()

        def body(g, carry):
            pltpu.sync_copy(half_v, acc.at[idx_d.at[g]], add=True)
            return carry

        lax.fori_loop(0, n_chunks, body, 0)
        plsc.subcore_barrier()
        pltpu.sync_copy(acc.at[pl.ds(s * rpt, rpt)], out_hbm.at[c, pl.ds(s * rpt, rpt)])

    return deg_kernel(dst_tiles, zeros_col)


def _sc_aggregate(h_split, src_tiles, dst_tiles, zeros_mat, n_pad, n_chunks):
    d2 = h_split.shape[2]
    rpt = n_pad // NS
    nbuf = 8 if d2 <= 32 else 6
    ga = nbuf - 3
    ss = 2

    @functools.partial(
        pl.kernel,
        mesh=_mesh(),
        compiler_params=pltpu.CompilerParams(use_tc_tiling_on_sc=False),
        out_type=jax.ShapeDtypeStruct((NC, n_pad, d2), jnp.float32),
        scratch_types=[
            pltpu.VMEM((n_chunks, K), jnp.int32),
            pltpu.VMEM((n_chunks, K), jnp.int32),
            pltpu.VMEM((nbuf, K, d2), jnp.float32),
            pltpu.VMEM_SHARED((n_pad, d2), jnp.float32),
            pltpu.SemaphoreType.DMA,
            pltpu.SemaphoreType.DMA,
            pltpu.SemaphoreType.DMA,
        ],
    )
    def agg_kernel(h_hbm, src_hbm, dst_hbm, z_hbm, out_hbm, idx_s, idx_d, rows, acc, gsem, ssem, isem):
        c = lax.axis_index("c")
        s = lax.axis_index("s")
        pltpu.async_copy(src_hbm.at[s], idx_s, isem)
        pltpu.async_copy(dst_hbm.at[s], idx_d, isem)
        pltpu.async_copy(
            z_hbm.at[pl.ds(s * rpt, rpt)], acc.at[pl.ds(s * rpt, rpt)], isem
        )
        pltpu.make_async_copy(src_hbm.at[s], idx_s, isem).wait()
        pltpu.make_async_copy(dst_hbm.at[s], idx_d, isem).wait()
        pltpu.make_async_copy(
            z_hbm.at[pl.ds(s * rpt, rpt)], acc.at[pl.ds(s * rpt, rpt)], isem
        ).wait()
        plsc.subcore_barrier()
        for j in range(ga):
            pltpu.async_copy(h_hbm.at[c].at[idx_s.at[j]], rows.at[j], gsem)

        def body(g, carry):
            slot = lax.rem(g, nbuf)
            pltpu.make_async_copy(h_hbm.at[c].at[idx_s.at[g]], rows.at[slot], gsem).wait()
            pltpu.async_copy(rows.at[slot], acc.at[idx_d.at[g]], ssem, add=True)

            @pl.when(g >= ss)
            def _drain_prev():
                pltpu.make_async_copy(
                    rows.at[lax.rem(g - ss, nbuf)], acc.at[idx_d.at[g - ss]], ssem
                ).wait()

            @pl.when(g + ga < n_chunks)
            def _prefetch():
                pltpu.async_copy(
                    h_hbm.at[c].at[idx_s.at[g + ga]],
                    rows.at[lax.rem(g + ga, nbuf)],
                    gsem,
                )

            return carry

        lax.fori_loop(0, n_chunks, body, 0)
        for j in range(n_chunks - ss, n_chunks):
            pltpu.make_async_copy(
                rows.at[j % nbuf], acc.at[idx_d.at[j]], ssem
            ).wait()
        plsc.subcore_barrier()
        pltpu.sync_copy(acc.at[pl.ds(s * rpt, rpt)], out_hbm.at[c, pl.ds(s * rpt, rpt)])

    return agg_kernel(h_split, src_tiles, dst_tiles, zeros_mat)


def _tc_mm_raw(x, ws, blk):
    n, din = x.shape
    d2 = ws.shape[2]

    def body(x_ref, w_ref, o_ref):
        o_ref[0] = jnp.dot(x_ref[...], w_ref[0], preferred_element_type=jnp.float32)

    return pl.pallas_call(
        body,
        grid=(NC, n // blk),
        in_specs=[
            pl.BlockSpec((blk, din), lambda t, i: (i, 0)),
            pl.BlockSpec((1, din, d2), lambda t, i: (t, 0, 0)),
        ],
        out_specs=pl.BlockSpec((1, blk, d2), lambda t, i: (t, i, 0)),
        out_shape=jax.ShapeDtypeStruct((NC, n, d2), jnp.float32),
    )(x, ws)


def _tc_scale(h_raw, d0, d1, blk):
    nc, n, d2 = h_raw.shape

    def body(h_ref, a_ref, b_ref, o_ref, s_ref):
        dis = lax.rsqrt(1.0 + a_ref[...] + b_ref[...])
        s_ref[...] = dis
        o_ref[...] = dis[None] * h_ref[...]

    return pl.pallas_call(
        body,
        grid=(n // blk,),
        in_specs=[
            pl.BlockSpec((NC, blk, d2), lambda i: (0, i, 0)),
            pl.BlockSpec((blk, 1), lambda i: (i, 0)),
            pl.BlockSpec((blk, 1), lambda i: (i, 0)),
        ],
        out_specs=[
            pl.BlockSpec((NC, blk, d2), lambda i: (0, i, 0)),
            pl.BlockSpec((blk, 1), lambda i: (i, 0)),
        ],
        out_shape=[
            jax.ShapeDtypeStruct((NC, n, d2), jnp.float32),
            jax.ShapeDtypeStruct((n, 1), jnp.float32),
        ],
    )(h_raw, d0, d1)


def _tc_mid(acc, ht, discol, b, w2s, blk):
    nc, n, d2 = ht.shape
    d = nc * d2
    do2 = w2s.shape[2]

    def body(a_ref, h_ref, s_ref, b_ref, w_ref, o_ref):
        agg = jnp.concatenate([a_ref[0] + h_ref[0], a_ref[1] + h_ref[1]], axis=-1)
        h1 = jnp.maximum(s_ref[...] * agg + b_ref[...], 0.0)
        o_ref[0] = s_ref[...] * jnp.dot(
            h1, w_ref[0], preferred_element_type=jnp.float32
        )

    return pl.pallas_call(
        body,
        grid=(NC, n // blk),
        in_specs=[
            pl.BlockSpec((NC, blk, d2), lambda t, i: (0, i, 0)),
            pl.BlockSpec((NC, blk, d2), lambda t, i: (0, i, 0)),
            pl.BlockSpec((blk, 1), lambda t, i: (i, 0)),
            pl.BlockSpec((1, d), lambda t, i: (0, 0)),
            pl.BlockSpec((1, d, do2), lambda t, i: (t, 0, 0)),
        ],
        out_specs=pl.BlockSpec((1, blk, do2), lambda t, i: (t, i, 0)),
        out_shape=jax.ShapeDtypeStruct((NC, n, do2), jnp.float32),
    )(acc, ht, discol, b, w2s)


def _tc_post(acc, ht, discol, b, blk):
    nc, n, d2 = ht.shape
    d = nc * d2

    def body(a_ref, h_ref, s_ref, b_ref, o_ref):
        agg = jnp.concatenate([a_ref[0] + h_ref[0], a_ref[1] + h_ref[1]], axis=-1)
        o_ref[...] = s_ref[...] * agg + b_ref[...]

    return pl.pallas_call(
        body,
        grid=(n // blk,),
        in_specs=[
            pl.BlockSpec((NC, blk, d2), lambda i: (0, i, 0)),
            pl.BlockSpec((NC, blk, d2), lambda i: (0, i, 0)),
            pl.BlockSpec((blk, 1), lambda i: (i, 0)),
            pl.BlockSpec((1, d), lambda i: (0, 0)),
        ],
        out_specs=pl.BlockSpec((blk, d), lambda i: (i, 0)),
        out_shape=jax.ShapeDtypeStruct((n, d), jnp.float32),
    )(acc, ht, discol, b)


def kernel(x, edge_index, W1, b1, W2, b2):
    n, din = x.shape
    e = edge_index.shape[1]

    epw = K * (-(-e // (K * NS)))
    n_chunks = epw // K
    e_pad = epw * NS
    n_pad = 128 * NS * (-(-(n + 1) // (128 * NS)))

    ei = edge_index.astype(jnp.int32)
    pad = e_pad - e
    src_t = jnp.concatenate([ei[0], jnp.zeros((pad,), jnp.int32)]).reshape(
        NS, n_chunks, K
    )
    dst_t = jnp.concatenate([ei[1], jnp.full((pad,), n, jnp.int32)]).reshape(
        NS, n_chunks, K
    )
    z_col = jnp.zeros((n_pad,), jnp.float32)
    z_hid = jnp.zeros((n_pad, W1.shape[1] // NC), jnp.float32)
    z_out = jnp.zeros((n_pad, W2.shape[1] // NC), jnp.float32)

    blk = 2000 if n % 2000 == 0 else n

    h2 = W1.shape[1] // NC
    o2 = W2.shape[1] // NC
    w1s = jnp.stack([W1[:, :h2], W1[:, h2:]])
    w2s = jnp.stack([W2[:, :o2], W2[:, o2:]])

    h1_raw = _tc_mm_raw(x, w1s, blk)
    deg = _sc_degree(dst_t, z_col, n_pad, n_chunks)
    ht1, discol = _tc_scale(
        h1_raw, deg[0, :n, None], deg[1, :n, None], blk
    )
    acc1 = _sc_aggregate(ht1, src_t, dst_t, z_hid, n_pad, n_chunks)
    ht2 = _tc_mid(acc1[:, :n], ht1, discol, b1[None, :], w2s, blk)
    acc2 = _sc_aggregate(ht2, src_t, dst_t, z_out, n_pad, n_chunks)
    return _tc_post(acc2[:, :n], ht2, discol, b2[None, :], blk)


# speedup vs baseline: 29.0587x; 1.0232x over previous
import functools

import jax
import jax.numpy as jnp
from jax import lax
from jax.experimental import pallas as pl
from jax.experimental.pallas import tpu as pltpu
from jax.experimental.pallas import tpu_sc as plsc

NC = 2
NS = 16
K = 128


def _mesh():
    return plsc.VectorSubcoreMesh(
        core_axis_name="c", subcore_axis_name="s", num_cores=NC, num_subcores=NS
    )


def _sc_degree(dst_tiles, zeros_col, n_pad, n_chunks):
    rpt = n_pad // NS

    @functools.partial(
        pl.kernel,
        mesh=_mesh(),
        compiler_params=pltpu.CompilerParams(use_tc_tiling_on_sc=False),
        out_type=jax.ShapeDtypeStruct((NC, n_pad), jnp.float32),
        scratch_types=[
            pltpu.VMEM((n_chunks, K), jnp.int32),
            pltpu.VMEM((K,), jnp.float32),
            pltpu.VMEM_SHARED((n_pad,), jnp.float32),
            pltpu.SemaphoreType.DMA,
        ],
    )
    def deg_kernel(dst_hbm, z_hbm, out_hbm, idx_d, half_v, acc, ssem):
        c = lax.axis_index("c")
        s = lax.axis_index("s")
        pltpu.sync_copy(dst_hbm.at[s], idx_d)
        for i in range(K // 16):
            half_v[pl.ds(16 * i, 16)] = jnp.full((16,), 0.5, jnp.float32)
        pltpu.sync_copy(z_hbm.at[pl.ds(s * rpt, rpt)], acc.at[pl.ds(s * rpt, rpt)])
        plsc.subcore_barrier()

        def body(g, carry):
            pltpu.async_copy(half_v, acc.at[idx_d.at[g]], ssem, add=True)

            @pl.when(g >= 4)
            def _drain():
                pltpu.make_async_copy(half_v, acc.at[idx_d.at[g - 4]], ssem).wait()

            return carry

        lax.fori_loop(0, n_chunks, body, 0)
        for j in range(n_chunks - 4, n_chunks):
            pltpu.make_async_copy(half_v, acc.at[idx_d.at[j]], ssem).wait()
        plsc.subcore_barrier()
        pltpu.sync_copy(acc.at[pl.ds(s * rpt, rpt)], out_hbm.at[c, pl.ds(s * rpt, rpt)])

    return deg_kernel(dst_tiles, zeros_col)


def _sc_aggregate(h_split, src_tiles, dst_tiles, zeros_mat, n_pad, n_chunks):
    d2 = h_split.shape[2]
    rpt = n_pad // NS
    nbuf = 8 if d2 <= 32 else 6
    ga = nbuf - 3
    ss = 2

    @functools.partial(
        pl.kernel,
        mesh=_mesh(),
        compiler_params=pltpu.CompilerParams(use_tc_tiling_on_sc=False),
        out_type=jax.ShapeDtypeStruct((NC, n_pad, d2), jnp.float32),
        scratch_types=[
            pltpu.VMEM((n_chunks, K), jnp.int32),
            pltpu.VMEM((n_chunks, K), jnp.int32),
            pltpu.VMEM((nbuf, K, d2), jnp.float32),
            pltpu.VMEM_SHARED((n_pad, d2), jnp.float32),
            pltpu.SemaphoreType.DMA,
            pltpu.SemaphoreType.DMA,
            pltpu.SemaphoreType.DMA,
        ],
    )
    def agg_kernel(h_hbm, src_hbm, dst_hbm, z_hbm, out_hbm, idx_s, idx_d, rows, acc, gsem, ssem, isem):
        c = lax.axis_index("c")
        s = lax.axis_index("s")
        pltpu.async_copy(src_hbm.at[s], idx_s, isem)
        pltpu.async_copy(dst_hbm.at[s], idx_d, isem)
        pltpu.async_copy(
            z_hbm.at[pl.ds(s * rpt, rpt)], acc.at[pl.ds(s * rpt, rpt)], isem
        )
        pltpu.make_async_copy(src_hbm.at[s], idx_s, isem).wait()
        pltpu.make_async_copy(dst_hbm.at[s], idx_d, isem).wait()
        pltpu.make_async_copy(
            z_hbm.at[pl.ds(s * rpt, rpt)], acc.at[pl.ds(s * rpt, rpt)], isem
        ).wait()
        plsc.subcore_barrier()
        for j in range(ga):
            pltpu.async_copy(h_hbm.at[c].at[idx_s.at[j]], rows.at[j], gsem)

        def body(g, carry):
            slot = lax.rem(g, nbuf)
            pltpu.make_async_copy(h_hbm.at[c].at[idx_s.at[g]], rows.at[slot], gsem).wait()
            pltpu.async_copy(rows.at[slot], acc.at[idx_d.at[g]], ssem, add=True)

            @pl.when(g >= ss)
            def _drain_prev():
                pltpu.make_async_copy(
                    rows.at[lax.rem(g - ss, nbuf)], acc.at[idx_d.at[g - ss]], ssem
                ).wait()

            @pl.when(g + ga < n_chunks)
            def _prefetch():
                pltpu.async_copy(
                    h_hbm.at[c].at[idx_s.at[g + ga]],
                    rows.at[lax.rem(g + ga, nbuf)],
                    gsem,
                )

            return carry

        lax.fori_loop(0, n_chunks, body, 0)
        for j in range(n_chunks - ss, n_chunks):
            pltpu.make_async_copy(
                rows.at[j % nbuf], acc.at[idx_d.at[j]], ssem
            ).wait()
        plsc.subcore_barrier()
        pltpu.sync_copy(acc.at[pl.ds(s * rpt, rpt)], out_hbm.at[c, pl.ds(s * rpt, rpt)])

    return agg_kernel(h_split, src_tiles, dst_tiles, zeros_mat)


def _tc_mm_raw(x, ws, blk):
    n, din = x.shape
    d2 = ws.shape[2]

    def body(x_ref, w_ref, o_ref):
        o_ref[0] = jnp.dot(x_ref[...], w_ref[0], preferred_element_type=jnp.float32)

    return pl.pallas_call(
        body,
        grid=(NC, n // blk),
        in_specs=[
            pl.BlockSpec((blk, din), lambda t, i: (i, 0)),
            pl.BlockSpec((1, din, d2), lambda t, i: (t, 0, 0)),
        ],
        out_specs=pl.BlockSpec((1, blk, d2), lambda t, i: (t, i, 0)),
        out_shape=jax.ShapeDtypeStruct((NC, n, d2), jnp.float32),
    )(x, ws)


def _tc_scale(h_raw, d0, d1, blk):
    nc, n, d2 = h_raw.shape

    def body(h_ref, a_ref, b_ref, o_ref, s_ref):
        dis = lax.rsqrt(1.0 + a_ref[...] + b_ref[...])
        s_ref[...] = dis
        o_ref[...] = dis[None] * h_ref[...]

    return pl.pallas_call(
        body,
        grid=(n // blk,),
        in_specs=[
            pl.BlockSpec((NC, blk, d2), lambda i: (0, i, 0)),
            pl.BlockSpec((blk, 1), lambda i: (i, 0)),
            pl.BlockSpec((blk, 1), lambda i: (i, 0)),
        ],
        out_specs=[
            pl.BlockSpec((NC, blk, d2), lambda i: (0, i, 0)),
            pl.BlockSpec((blk, 1), lambda i: (i, 0)),
        ],
        out_shape=[
            jax.ShapeDtypeStruct((NC, n, d2), jnp.float32),
            jax.ShapeDtypeStruct((n, 1), jnp.float32),
        ],
    )(h_raw, d0, d1)


def _tc_mid(acc, ht, discol, b, w2s, blk):
    nc, n, d2 = ht.shape
    d = nc * d2
    do2 = w2s.shape[2]

    def body(a_ref, h_ref, s_ref, b_ref, w_ref, o_ref):
        agg = jnp.concatenate([a_ref[0] + h_ref[0], a_ref[1] + h_ref[1]], axis=-1)
        h1 = jnp.maximum(s_ref[...] * agg + b_ref[...], 0.0)
        o_ref[0] = s_ref[...] * jnp.dot(
            h1, w_ref[0], preferred_element_type=jnp.float32
        )

    return pl.pallas_call(
        body,
        grid=(NC, n // blk),
        in_specs=[
            pl.BlockSpec((NC, blk, d2), lambda t, i: (0, i, 0)),
            pl.BlockSpec((NC, blk, d2), lambda t, i: (0, i, 0)),
            pl.BlockSpec((blk, 1), lambda t, i: (i, 0)),
            pl.BlockSpec((1, d), lambda t, i: (0, 0)),
            pl.BlockSpec((1, d, do2), lambda t, i: (t, 0, 0)),
        ],
        out_specs=pl.BlockSpec((1, blk, do2), lambda t, i: (t, i, 0)),
        out_shape=jax.ShapeDtypeStruct((NC, n, do2), jnp.float32),
    )(acc, ht, discol, b, w2s)


def _tc_post(acc, ht, discol, b, blk):
    nc, n, d2 = ht.shape
    d = nc * d2

    def body(a_ref, h_ref, s_ref, b_ref, o_ref):
        agg = jnp.concatenate([a_ref[0] + h_ref[0], a_ref[1] + h_ref[1]], axis=-1)
        o_ref[...] = s_ref[...] * agg + b_ref[...]

    return pl.pallas_call(
        body,
        grid=(n // blk,),
        in_specs=[
            pl.BlockSpec((NC, blk, d2), lambda i: (0, i, 0)),
            pl.BlockSpec((NC, blk, d2), lambda i: (0, i, 0)),
            pl.BlockSpec((blk, 1), lambda i: (i, 0)),
            pl.BlockSpec((1, d), lambda i: (0, 0)),
        ],
        out_specs=pl.BlockSpec((blk, d), lambda i: (i, 0)),
        out_shape=jax.ShapeDtypeStruct((n, d), jnp.float32),
    )(acc, ht, discol, b)


def kernel(x, edge_index, W1, b1, W2, b2):
    n, din = x.shape
    e = edge_index.shape[1]

    epw = K * (-(-e // (K * NS)))
    n_chunks = epw // K
    e_pad = epw * NS
    n_pad = 128 * NS * (-(-(n + 1) // (128 * NS)))

    ei = edge_index.astype(jnp.int32)
    pad = e_pad - e
    src_t = jnp.concatenate([ei[0], jnp.zeros((pad,), jnp.int32)]).reshape(
        NS, n_chunks, K
    )
    dst_t = jnp.concatenate([ei[1], jnp.full((pad,), n, jnp.int32)]).reshape(
        NS, n_chunks, K
    )
    z_col = jnp.zeros((n_pad,), jnp.float32)
    z_hid = jnp.zeros((n_pad, W1.shape[1] // NC), jnp.float32)
    z_out = jnp.zeros((n_pad, W2.shape[1] // NC), jnp.float32)

    blk = 2000 if n % 2000 == 0 else n

    h2 = W1.shape[1] // NC
    o2 = W2.shape[1] // NC
    w1s = jnp.stack([W1[:, :h2], W1[:, h2:]])
    w2s = jnp.stack([W2[:, :o2], W2[:, o2:]])

    h1_raw = _tc_mm_raw(x, w1s, blk)
    deg = _sc_degree(dst_t, z_col, n_pad, n_chunks)
    ht1, discol = _tc_scale(
        h1_raw, deg[0, :n, None], deg[1, :n, None], blk
    )
    acc1 = _sc_aggregate(ht1, src_t, dst_t, z_hid, n_pad, n_chunks)
    ht2 = _tc_mid(acc1[:, :n], ht1, discol, b1[None, :], w2s, blk)
    acc2 = _sc_aggregate(ht2, src_t, dst_t, z_out, n_pad, n_chunks)
    return _tc_post(acc2[:, :n], ht2, discol, b2[None, :], blk)
